# baseline (speedup 1.0000x reference)
import jax
import jax.numpy as jnp
from jax import lax
from jax.experimental import pallas as pl
from jax.experimental.pallas import tpu as pltpu
from jax.experimental.pallas import tpu_sc as plsc

N_USER = 50000
N_ITEM = 50000
EMB = 32
N_EDGES = 1600000

NT = 16
LANES = 16
N_PAD = 51200
NODES_PER_TILE = N_PAD // NT
WB_CHUNK = 32
CHUNK = 128
NSLOT = 4
LOOKAHEAD = 2
SG = 8
DEG_GROUP = 8
N_CHUNKS = 12544
E_PAD = N_CHUNKS * CHUNK
CHUNKS_PER_TILE = N_CHUNKS // NT
SUPERS_PER_TILE = CHUNKS_PER_TILE // SG
ITERS_PER_TILE = CHUNKS_PER_TILE // NSLOT
DEG_GROUPS_PER_TILE = CHUNKS_PER_TILE // DEG_GROUP
PAD_NODE = N_PAD - 1

_mesh = plsc.VectorSubcoreMesh(core_axis_name="c", subcore_axis_name="s")

_f32 = jnp.float32
_zeros16 = lambda: jnp.zeros((LANES,), _f32)


def _deg_body(edges2, recs_out, idx_buf, ones_buf, red_buf, out_buf,
              deg_acc, sem_s):
    c = lax.axis_index("c")
    s = lax.axis_index("s")
    nb = s * NODES_PER_TILE
    cnt = 1 - c

    def fill_ones(j, carry):
        ones_buf[pl.ds(j * LANES, LANES)] = jnp.ones((LANES,), _f32)
        return carry
    lax.fori_loop(0, CHUNK // LANES, fill_ones, 0)

    def fill_zero(j, carry):
        out_buf[pl.ds(j * LANES, LANES)] = _zeros16()
        return carry
    lax.fori_loop(0, NODES_PER_TILE // LANES, fill_zero, 0)
    pltpu.sync_copy(out_buf, deg_acc.at[pl.ds(nb, NODES_PER_TILE)])
    plsc.subcore_barrier()

    def group_body(g, carry):
        base = s * CHUNKS_PER_TILE + g * DEG_GROUP
        pltpu.sync_copy(edges2.at[cnt, pl.ds(base, DEG_GROUP)], idx_buf)
        cps = [pltpu.async_copy(ones_buf, deg_acc.at[idx_buf.at[j]],
                                sem_s, add=True)
               for j in range(DEG_GROUP)]
        for cp in cps:
            cp.wait()
        return carry
    lax.fori_loop(0, DEG_GROUPS_PER_TILE, group_body, 0)
    plsc.subcore_barrier()

    pltpu.sync_copy(deg_acc.at[pl.ds(nb, NODES_PER_TILE)], red_buf)

    def red_body(j, carry):
        tot = red_buf[pl.ds(j * LANES, LANES)]
        out_buf[pl.ds(j * LANES, LANES)] = 1.0 / jnp.maximum(tot, 1.0)
        return carry
    lax.fori_loop(0, NODES_PER_TILE // LANES, red_body, 0)
    pltpu.sync_copy(out_buf, recs_out.at[pl.ds(cnt * N_PAD + nb, NODES_PER_TILE)])


def _prop_body(*refs):
        (edges2, tabs, outs,
         acc, gidx3, sidx3, rows,
         sem_i, *sems) = refs
        sems_g = sems[:NSLOT]
        sems_s = sems[NSLOT:]

        c = lax.axis_index("c")
        s = lax.axis_index("s")
        gd = c
        sd = 1 - c

        def zb(n, carry):
            rows[0, n, pl.ds(0, LANES)] = _zeros16()
            rows[0, n, pl.ds(LANES, LANES)] = _zeros16()
            return carry
        lax.fori_loop(0, CHUNK, zb, 0)

        nzc = NODES_PER_TILE // CHUNK
        zcps = []

        def zacc(q, carry):
            pltpu.make_async_copy(
                rows.at[0],
                acc.at[pl.ds(s * NODES_PER_TILE + q * CHUNK, CHUNK)],
                sems_s[0]).start()
            return carry
        lax.fori_loop(0, nzc, zacc, 0)

        def zacc_wait(q, carry):
            pltpu.make_async_copy(
                rows.at[0],
                acc.at[pl.ds(s * NODES_PER_TILE + q * CHUNK, CHUNK)],
                sems_s[0]).wait()
            return carry
        lax.fori_loop(0, nzc, zacc_wait, 0)
        plsc.subcore_barrier()

        base_chunk = s * CHUNKS_PER_TILE

        def stage_desc(sg, arr, buf3):
            return pltpu.make_async_copy(
                edges2.at[arr, pl.ds(base_chunk + sg * SG, SG)],
                buf3.at[lax.rem(sg, 3)], sem_i)

        def stage(sg):
            stage_desc(sg, gd, gidx3).start()
            stage_desc(sg, sd, sidx3).start()

        def wait_stage(sg):
            stage_desc(sg, gd, gidx3).wait()
            stage_desc(sg, sd, sidx3).wait()

        def gather_desc(t, slot):
            sgs = lax.rem(t // SG, 3)
            jof = lax.rem(t, SG)
            return pltpu.make_async_copy(
                tabs.at[gd].at[gidx3.at[sgs, jof]], rows.at[slot],
                sems_g[slot])

        def scatter_desc(t, slot):
            sgs = lax.rem(t // SG, 3)
            jof = lax.rem(t, SG)
            return pltpu.make_async_copy(
                rows.at[slot], acc.at[sidx3.at[sgs, jof]], sems_s[slot])

        pltpu.sync_copy(edges2.at[gd, pl.ds(base_chunk, SG)], gidx3.at[0])
        pltpu.sync_copy(edges2.at[sd, pl.ds(base_chunk, SG)], sidx3.at[0])
        stage(1)
        for t0 in range(LOOKAHEAD):
            gather_desc(t0, t0).start()

        def outer(it, carry):
            for k in range(NSLOT):
                t = it * NSLOT + k
                tg = t + LOOKAHEAD
                gslot = (k + LOOKAHEAD) % NSLOT
                if k == 2:
                    def do_stage(tg=tg):
                        sgg = tg // SG
                        wait_stage(sgg)
                        pl.when(sgg + 1 < SUPERS_PER_TILE)(
                            lambda: stage(sgg + 1))
                    pl.when(jnp.logical_and(lax.rem(tg, SG) == 0,
                                            tg < CHUNKS_PER_TILE))(do_stage)
                pl.when(t >= LOOKAHEAD)(
                    lambda t=t, gs=gslot: scatter_desc(t - LOOKAHEAD,
                                                       gs).wait())
                pl.when(tg < CHUNKS_PER_TILE)(
                    lambda tg=tg, gs=gslot: gather_desc(tg, gs).start())
                gather_desc(t, k).wait()
                scatter_desc(t, k).start(add=True)
            return carry
        lax.fori_loop(0, ITERS_PER_TILE, outer, 0)

        for t_tail in range(CHUNKS_PER_TILE - LOOKAHEAD, CHUNKS_PER_TILE):
            scatter_desc(t_tail, t_tail % NSLOT).wait()
        plsc.subcore_barrier()

        nb0 = s * NODES_PER_TILE
        pltpu.sync_copy(acc.at[pl.ds(nb0, NODES_PER_TILE)],
                        outs.at[sd, pl.ds(nb0, NODES_PER_TILE)])


_deg_kernel = pl.kernel(
    _deg_body,
    out_type=jax.ShapeDtypeStruct((2 * N_PAD,), _f32),
    mesh=_mesh,
    compiler_params=pltpu.CompilerParams(use_tc_tiling_on_sc=False),
    scratch_types=[
        pltpu.VMEM((DEG_GROUP, CHUNK), jnp.int32),
        pltpu.VMEM((CHUNK,), jnp.float32),
        pltpu.VMEM((NODES_PER_TILE,), jnp.float32),
        pltpu.VMEM((NODES_PER_TILE,), jnp.float32),
        pltpu.VMEM_SHARED((N_PAD,), jnp.float32),
        pltpu.SemaphoreType.DMA,
    ],
)

_prop_scratch = [
    pltpu.VMEM_SHARED((N_PAD, EMB), jnp.float32),
    pltpu.VMEM((3, SG, CHUNK), jnp.int32),
    pltpu.VMEM((3, SG, CHUNK), jnp.int32),
    pltpu.VMEM((NSLOT, CHUNK, EMB), jnp.float32),
]
_prop_sems = [pltpu.SemaphoreType.DMA] * (1 + 2 * NSLOT)

_prop_kernel = pl.kernel(
    _prop_body,
    out_type=jax.ShapeDtypeStruct((2, N_PAD, EMB), _f32),
    mesh=_mesh,
    compiler_params=pltpu.CompilerParams(use_tc_tiling_on_sc=False),
    scratch_types=_prop_scratch + _prop_sems,
)

TBLK = 2048
_TGRID = (2 * N_PAD) // TBLK


def _scale_body(x_ref, r_ref, o_ref):
    o_ref[...] = x_ref[...] * r_ref[...]


_scale_tc = pl.pallas_call(
    _scale_body,
    out_shape=jax.ShapeDtypeStruct((2 * N_PAD, EMB), _f32),
    grid=(_TGRID,),
    in_specs=[pl.BlockSpec((TBLK, EMB), lambda i: (i, 0)),
              pl.BlockSpec((TBLK, 1), lambda i: (i, 0))],
    out_specs=pl.BlockSpec((TBLK, EMB), lambda i: (i, 0)),
)


def _final_body(x_ref, r_ref, h0_ref, h1_ref, o_ref):
    o_ref[...] = (h0_ref[...] + h1_ref[...]
                  + x_ref[...] * r_ref[...]) * _f32(1.0 / 3.0)


_final_tc = pl.pallas_call(
    _final_body,
    out_shape=jax.ShapeDtypeStruct((2 * N_PAD, EMB), _f32),
    grid=(_TGRID,),
    in_specs=[pl.BlockSpec((TBLK, EMB), lambda i: (i, 0)),
              pl.BlockSpec((TBLK, 1), lambda i: (i, 0)),
              pl.BlockSpec((TBLK, EMB), lambda i: (i, 0)),
              pl.BlockSpec((TBLK, EMB), lambda i: (i, 0))],
    out_specs=pl.BlockSpec((TBLK, EMB), lambda i: (i, 0)),
)


def kernel(edge_index, user_emb, item_emb):
    pad = jnp.full((2, E_PAD - N_EDGES), PAD_NODE, jnp.int32)
    edges2 = jnp.concatenate([edge_index, pad], axis=1)
    edges2 = edges2.reshape(2, N_CHUNKS, CHUNK)
    zrow = jnp.zeros((1, N_PAD - N_USER, EMB), _f32)
    tabs0 = jnp.concatenate(
        [jnp.stack([user_emb, item_emb]),
         jnp.concatenate([zrow, zrow])], axis=1)

    recs = _deg_kernel(edges2)
    rec2d = recs.reshape(2 * N_PAD, 1)
    sums1 = _prop_kernel(edges2, tabs0)
    tabs1 = _scale_tc(sums1.reshape(2 * N_PAD, EMB), rec2d)
    sums2 = _prop_kernel(edges2, tabs1.reshape(2, N_PAD, EMB))
    outs = _final_tc(sums2.reshape(2 * N_PAD, EMB), rec2d,
                     tabs0.reshape(2 * N_PAD, EMB), tabs1)
    outs = outs.reshape(2, N_PAD, EMB)
    return outs[0, :N_USER], outs[1, :N_ITEM]


# speedup vs baseline: 1.0425x; 1.0425x over previous
import jax
import jax.numpy as jnp
from jax import lax
from jax.experimental import pallas as pl
from jax.experimental.pallas import tpu as pltpu
from jax.experimental.pallas import tpu_sc as plsc

N_USER = 50000
N_ITEM = 50000
EMB = 32
N_EDGES = 1600000

NT = 16
LANES = 16
N_PAD = 51200
NODES_PER_TILE = N_PAD // NT
CHUNK = 128
NSLOT = 4
LOOKAHEAD = 2
SG = 8
N_CHUNKS = 12544
E_PAD = N_CHUNKS * CHUNK
CHUNKS_PER_TILE = N_CHUNKS // NT
SUPERS_PER_TILE = CHUNKS_PER_TILE // SG
ITERS_PER_TILE = CHUNKS_PER_TILE // NSLOT
PAD_NODE = N_PAD - 1
DEG_BUF = NODES_PER_TILE // 2

_mesh = plsc.VectorSubcoreMesh(core_axis_name="c", subcore_axis_name="s")

_f32 = jnp.float32
_zeros16 = lambda: jnp.zeros((LANES,), _f32)


def _make_prop_body(with_deg):
    def body(*refs):
        if with_deg:
            (edges2, tabs, outs, recs,
             acc, hist, gidx3, sidx3, rows, ones_buf, dbuf,
             sem_i, *sems) = refs
            sems_g = sems[:NSLOT]
            sems_s = sems[NSLOT:2 * NSLOT]
            sems_o = sems[2 * NSLOT:]
        else:
            (edges2, tabs, outs,
             acc, gidx3, sidx3, rows,
             sem_i, *sems) = refs
            sems_g = sems[:NSLOT]
            sems_s = sems[NSLOT:]

        c = lax.axis_index("c")
        s = lax.axis_index("s")
        gd = c
        sd = 1 - c

        def zb(n, carry):
            rows[0, n, pl.ds(0, LANES)] = _zeros16()
            rows[0, n, pl.ds(LANES, LANES)] = _zeros16()
            return carry
        lax.fori_loop(0, CHUNK, zb, 0)

        nzc = NODES_PER_TILE // CHUNK
        def zacc(q, carry):
            pltpu.make_async_copy(
                rows.at[0],
                acc.at[pl.ds(s * NODES_PER_TILE + q * CHUNK, CHUNK)],
                sems_s[0]).start()
            return carry
        lax.fori_loop(0, nzc, zacc, 0)

        def zacc_wait(q, carry):
            pltpu.make_async_copy(
                rows.at[0],
                acc.at[pl.ds(s * NODES_PER_TILE + q * CHUNK, CHUNK)],
                sems_s[0]).wait()
            return carry
        lax.fori_loop(0, nzc, zacc_wait, 0)

        if with_deg:
            def zdeg(j, carry):
                dbuf[pl.ds(j * LANES, LANES)] = _zeros16()
                return carry
            lax.fori_loop(0, DEG_BUF // LANES, zdeg, 0)
            for h in range(NODES_PER_TILE // DEG_BUF):
                pltpu.sync_copy(
                    dbuf,
                    hist.at[pl.ds(s * NODES_PER_TILE + h * DEG_BUF,
                                  DEG_BUF)])

            def fill_ones(j, carry):
                ones_buf[pl.ds(j * LANES, LANES)] = jnp.ones((LANES,), _f32)
                return carry
            lax.fori_loop(0, CHUNK // LANES, fill_ones, 0)
        plsc.subcore_barrier()

        base_chunk = s * CHUNKS_PER_TILE

        def stage_desc(sg, arr, buf3):
            return pltpu.make_async_copy(
                edges2.at[arr, pl.ds(base_chunk + sg * SG, SG)],
                buf3.at[lax.rem(sg, 3)], sem_i)

        def stage(sg):
            stage_desc(sg, gd, gidx3).start()
            stage_desc(sg, sd, sidx3).start()

        def wait_stage(sg):
            stage_desc(sg, gd, gidx3).wait()
            stage_desc(sg, sd, sidx3).wait()

        def gather_desc(t, slot):
            sgs = lax.rem(t // SG, 3)
            jof = lax.rem(t, SG)
            return pltpu.make_async_copy(
                tabs.at[gd].at[gidx3.at[sgs, jof]], rows.at[slot],
                sems_g[slot])

        def scatter_desc(t, slot):
            sgs = lax.rem(t // SG, 3)
            jof = lax.rem(t, SG)
            return pltpu.make_async_copy(
                rows.at[slot], acc.at[sidx3.at[sgs, jof]], sems_s[slot])

        if with_deg:
            def ones_desc(t, slot):
                sgs = lax.rem(t // SG, 3)
                jof = lax.rem(t, SG)
                return pltpu.make_async_copy(
                    ones_buf, hist.at[sidx3.at[sgs, jof]], sems_o[slot])

        pltpu.sync_copy(edges2.at[gd, pl.ds(base_chunk, SG)], gidx3.at[0])
        pltpu.sync_copy(edges2.at[sd, pl.ds(base_chunk, SG)], sidx3.at[0])
        stage(1)
        for t0 in range(LOOKAHEAD):
            gather_desc(t0, t0).start()

        def outer(it, carry):
            for k in range(NSLOT):
                t = it * NSLOT + k
                tg = t + LOOKAHEAD
                gslot = (k + LOOKAHEAD) % NSLOT
                if k == 2:
                    def do_stage(tg=tg):
                        sgg = tg // SG
                        wait_stage(sgg)
                        pl.when(sgg + 1 < SUPERS_PER_TILE)(
                            lambda: stage(sgg + 1))
                    pl.when(jnp.logical_and(lax.rem(tg, SG) == 0,
                                            tg < CHUNKS_PER_TILE))(do_stage)

                def drain(t=t, gs=gslot):
                    scatter_desc(t - LOOKAHEAD, gs).wait()
                    if with_deg:
                        ones_desc(t - LOOKAHEAD, gs).wait()
                pl.when(t >= LOOKAHEAD)(drain)
                pl.when(tg < CHUNKS_PER_TILE)(
                    lambda tg=tg, gs=gslot: gather_desc(tg, gs).start())
                gather_desc(t, k).wait()
                scatter_desc(t, k).start(add=True)
                if with_deg:
                    ones_desc(t, k).start(add=True)
            return carry
        lax.fori_loop(0, ITERS_PER_TILE, outer, 0)

        for t_tail in range(CHUNKS_PER_TILE - LOOKAHEAD, CHUNKS_PER_TILE):
            scatter_desc(t_tail, t_tail % NSLOT).wait()
            if with_deg:
                ones_desc(t_tail, t_tail % NSLOT).wait()
        plsc.subcore_barrier()

        nb0 = s * NODES_PER_TILE
        pltpu.sync_copy(acc.at[pl.ds(nb0, NODES_PER_TILE)],
                        outs.at[sd, pl.ds(nb0, NODES_PER_TILE)])

        if with_deg:
            for h in range(NODES_PER_TILE // DEG_BUF):
                pltpu.sync_copy(
                    hist.at[pl.ds(nb0 + h * DEG_BUF, DEG_BUF)], dbuf)

                def inv(j, carry):
                    tot = dbuf[pl.ds(j * LANES, LANES)]
                    dbuf[pl.ds(j * LANES, LANES)] = (
                        1.0 / jnp.maximum(tot, 1.0))
                    return carry
                lax.fori_loop(0, DEG_BUF // LANES, inv, 0)
                pltpu.sync_copy(
                    dbuf,
                    recs.at[pl.ds(sd * N_PAD + nb0 + h * DEG_BUF, DEG_BUF)])

    return body


_prop1_kernel = pl.kernel(
    _make_prop_body(True),
    out_type=(jax.ShapeDtypeStruct((2, N_PAD, EMB), _f32),
              jax.ShapeDtypeStruct((2 * N_PAD,), _f32)),
    mesh=_mesh,
    compiler_params=pltpu.CompilerParams(use_tc_tiling_on_sc=False),
    scratch_types=[
        pltpu.VMEM_SHARED((N_PAD, EMB), jnp.float32),
        pltpu.VMEM_SHARED((N_PAD,), jnp.float32),
        pltpu.VMEM((3, SG, CHUNK), jnp.int32),
        pltpu.VMEM((3, SG, CHUNK), jnp.int32),
        pltpu.VMEM((NSLOT, CHUNK, EMB), jnp.float32),
        pltpu.VMEM((CHUNK,), jnp.float32),
        pltpu.VMEM((DEG_BUF,), jnp.float32),
    ] + [pltpu.SemaphoreType.DMA] * (1 + 3 * NSLOT),
)

_prop2_kernel = pl.kernel(
    _make_prop_body(False),
    out_type=jax.ShapeDtypeStruct((2, N_PAD, EMB), _f32),
    mesh=_mesh,
    compiler_params=pltpu.CompilerParams(use_tc_tiling_on_sc=False),
    scratch_types=[
        pltpu.VMEM_SHARED((N_PAD, EMB), jnp.float32),
        pltpu.VMEM((3, SG, CHUNK), jnp.int32),
        pltpu.VMEM((3, SG, CHUNK), jnp.int32),
        pltpu.VMEM((NSLOT, CHUNK, EMB), jnp.float32),
    ] + [pltpu.SemaphoreType.DMA] * (1 + 2 * NSLOT),
)

TBLK = 2048
_TGRID = (2 * N_PAD) // TBLK


def _scale_body(x_ref, r_ref, o_ref):
    o_ref[...] = x_ref[...] * r_ref[...]


_scale_tc = pl.pallas_call(
    _scale_body,
    out_shape=jax.ShapeDtypeStruct((2 * N_PAD, EMB), _f32),
    grid=(_TGRID,),
    in_specs=[pl.BlockSpec((TBLK, EMB), lambda i: (i, 0)),
              pl.BlockSpec((TBLK, 1), lambda i: (i, 0))],
    out_specs=pl.BlockSpec((TBLK, EMB), lambda i: (i, 0)),
)

FBLK = 400
_FGRID = N_USER // FBLK
_FOFF = N_PAD // FBLK


def _final_body(xu_ref, xi_ref, ru_ref, ri_ref, h0u_ref, h0i_ref,
                h1u_ref, h1i_ref, ou_ref, oi_ref):
    third = _f32(1.0 / 3.0)
    ou_ref[...] = (h0u_ref[...] + h1u_ref[...]
                   + xu_ref[...] * ru_ref[...]) * third
    oi_ref[...] = (h0i_ref[...] + h1i_ref[...]
                   + xi_ref[...] * ri_ref[...]) * third


def _fspec(off, ncol):
    return pl.BlockSpec((FBLK, ncol), lambda i, off=off: (i + off, 0))


_final_tc = pl.pallas_call(
    _final_body,
    out_shape=(jax.ShapeDtypeStruct((N_USER, EMB), _f32),
               jax.ShapeDtypeStruct((N_ITEM, EMB), _f32)),
    grid=(_FGRID,),
    in_specs=[_fspec(0, EMB), _fspec(_FOFF, EMB),
              _fspec(0, 1), _fspec(_FOFF, 1),
              _fspec(0, EMB), _fspec(_FOFF, EMB),
              _fspec(0, EMB), _fspec(_FOFF, EMB)],
    out_specs=(_fspec(0, EMB), _fspec(0, EMB)),
)


def kernel(edge_index, user_emb, item_emb):
    pad = jnp.full((2, E_PAD - N_EDGES), PAD_NODE, jnp.int32)
    edges2 = jnp.concatenate([edge_index, pad], axis=1)
    edges2 = edges2.reshape(2, N_CHUNKS, CHUNK)
    zrow = jnp.zeros((1, N_PAD - N_USER, EMB), _f32)
    tabs0 = jnp.concatenate(
        [jnp.stack([user_emb, item_emb]),
         jnp.concatenate([zrow, zrow])], axis=1)

    sums1, recs = _prop1_kernel(edges2, tabs0)
    rec2d = recs.reshape(2 * N_PAD, 1)
    tabs1 = _scale_tc(sums1.reshape(2 * N_PAD, EMB), rec2d)
    sums2 = _prop2_kernel(edges2, tabs1.reshape(2, N_PAD, EMB))
    sums2f = sums2.reshape(2 * N_PAD, EMB)
    tabs0f = tabs0.reshape(2 * N_PAD, EMB)
    out_u, out_i = _final_tc(sums2f, sums2f, rec2d, rec2d,
                             tabs0f, tabs0f, tabs1, tabs1)
    return out_u, out_i


# speedup vs baseline: 1.0572x; 1.0141x over previous
import jax
import jax.numpy as jnp
from jax import lax
from jax.experimental import pallas as pl
from jax.experimental.pallas import tpu as pltpu
from jax.experimental.pallas import tpu_sc as plsc

N_USER = 50000
N_ITEM = 50000
EMB = 32
N_EDGES = 1600000

NT = 16
LANES = 16
N_PAD = 51200
NODES_PER_TILE = N_PAD // NT
CHUNK = 128
NSLOT = 4
LOOKAHEAD = 2
SG = 8
N_CHUNKS = 12544
E_PAD = N_CHUNKS * CHUNK
CHUNKS_PER_TILE = N_CHUNKS // NT
SUPERS_PER_TILE = CHUNKS_PER_TILE // SG
ITERS_PER_TILE = CHUNKS_PER_TILE // NSLOT
PAD_NODE = N_PAD - 1
DEG_BUF = NODES_PER_TILE // 2

_mesh = plsc.VectorSubcoreMesh(core_axis_name="c", subcore_axis_name="s")

_f32 = jnp.float32
_zeros16 = lambda: jnp.zeros((LANES,), _f32)


def _make_prop_body(with_deg):
    def body(*refs):
        if with_deg:
            (edges2, tabs, outs, recs,
             acc, hist, gidx3, sidx3, rows, ones_buf, dbuf,
             sem_i, *sems) = refs
            sems_g = sems[:NSLOT]
            sems_s = sems[NSLOT:2 * NSLOT]
            sems_o = sems[2 * NSLOT:]
        else:
            (edges2, tabs, outs,
             acc, gidx3, sidx3, rows,
             sem_i, *sems) = refs
            sems_g = sems[:NSLOT]
            sems_s = sems[NSLOT:]

        c = lax.axis_index("c")
        s = lax.axis_index("s")
        gd = c
        sd = 1 - c

        def zb(n, carry):
            rows[0, n, pl.ds(0, LANES)] = _zeros16()
            rows[0, n, pl.ds(LANES, LANES)] = _zeros16()
            return carry
        lax.fori_loop(0, CHUNK, zb, 0)

        nzc = NODES_PER_TILE // CHUNK
        def zacc(q, carry):
            pltpu.make_async_copy(
                rows.at[0],
                acc.at[pl.ds(s * NODES_PER_TILE + q * CHUNK, CHUNK)],
                sems_s[0]).start()
            return carry
        lax.fori_loop(0, nzc, zacc, 0)

        def zacc_wait(q, carry):
            pltpu.make_async_copy(
                rows.at[0],
                acc.at[pl.ds(s * NODES_PER_TILE + q * CHUNK, CHUNK)],
                sems_s[0]).wait()
            return carry
        lax.fori_loop(0, nzc, zacc_wait, 0)

        if with_deg:
            def zdeg(j, carry):
                dbuf[pl.ds(j * LANES, LANES)] = _zeros16()
                return carry
            lax.fori_loop(0, DEG_BUF // LANES, zdeg, 0)
            for h in range(NODES_PER_TILE // DEG_BUF):
                pltpu.sync_copy(
                    dbuf,
                    hist.at[pl.ds(s * NODES_PER_TILE + h * DEG_BUF,
                                  DEG_BUF)])

            def fill_ones(j, carry):
                ones_buf[pl.ds(j * LANES, LANES)] = jnp.ones((LANES,), _f32)
                return carry
            lax.fori_loop(0, CHUNK // LANES, fill_ones, 0)
        plsc.subcore_barrier()

        base_chunk = s * CHUNKS_PER_TILE

        def stage_desc(sg, arr, buf3):
            return pltpu.make_async_copy(
                edges2.at[arr, pl.ds(base_chunk + sg * SG, SG)],
                buf3.at[lax.rem(sg, 3)], sem_i)

        def stage(sg):
            stage_desc(sg, gd, gidx3).start()
            stage_desc(sg, sd, sidx3).start()

        def wait_stage(sg):
            stage_desc(sg, gd, gidx3).wait()
            stage_desc(sg, sd, sidx3).wait()

        def gather_desc(t, slot):
            sgs = lax.rem(t // SG, 3)
            jof = lax.rem(t, SG)
            return pltpu.make_async_copy(
                tabs.at[gd].at[gidx3.at[sgs, jof]], rows.at[slot],
                sems_g[slot])

        def scatter_desc(t, slot):
            sgs = lax.rem(t // SG, 3)
            jof = lax.rem(t, SG)
            return pltpu.make_async_copy(
                rows.at[slot], acc.at[sidx3.at[sgs, jof]], sems_s[slot])

        if with_deg:
            def ones_desc(t, slot):
                sgs = lax.rem(t // SG, 3)
                jof = lax.rem(t, SG)
                return pltpu.make_async_copy(
                    ones_buf, hist.at[sidx3.at[sgs, jof]], sems_o[slot])

        pltpu.sync_copy(edges2.at[gd, pl.ds(base_chunk, SG)], gidx3.at[0])
        pltpu.sync_copy(edges2.at[sd, pl.ds(base_chunk, SG)], sidx3.at[0])
        stage(1)
        for t0 in range(LOOKAHEAD):
            gather_desc(t0, t0).start()

        def outer(it, carry):
            for k in range(NSLOT):
                t = it * NSLOT + k
                tg = t + LOOKAHEAD
                gslot = (k + LOOKAHEAD) % NSLOT
                if k == 2:
                    def do_stage(tg=tg):
                        sgg = tg // SG
                        wait_stage(sgg)
                        pl.when(sgg + 1 < SUPERS_PER_TILE)(
                            lambda: stage(sgg + 1))
                    pl.when(jnp.logical_and(lax.rem(tg, SG) == 0,
                                            tg < CHUNKS_PER_TILE))(do_stage)

                def drain(t=t, gs=gslot):
                    scatter_desc(t - LOOKAHEAD, gs).wait()
                    if with_deg:
                        ones_desc(t - LOOKAHEAD, gs).wait()
                pl.when(t >= LOOKAHEAD)(drain)
                pl.when(tg < CHUNKS_PER_TILE)(
                    lambda tg=tg, gs=gslot: gather_desc(tg, gs).start())
                gather_desc(t, k).wait()
                scatter_desc(t, k).start(add=True)
                if with_deg:
                    ones_desc(t, k).start(add=True)
            return carry
        lax.fori_loop(0, ITERS_PER_TILE, outer, 0)

        for t_tail in range(CHUNKS_PER_TILE - LOOKAHEAD, CHUNKS_PER_TILE):
            scatter_desc(t_tail, t_tail % NSLOT).wait()
            if with_deg:
                ones_desc(t_tail, t_tail % NSLOT).wait()
        plsc.subcore_barrier()

        nb0 = s * NODES_PER_TILE
        pltpu.sync_copy(acc.at[pl.ds(nb0, NODES_PER_TILE)],
                        outs.at[sd, pl.ds(nb0, NODES_PER_TILE)])

        if with_deg:
            for h in range(NODES_PER_TILE // DEG_BUF):
                pltpu.sync_copy(
                    hist.at[pl.ds(nb0 + h * DEG_BUF, DEG_BUF)], dbuf)

                def inv(j, carry):
                    tot = dbuf[pl.ds(j * LANES, LANES)]
                    dbuf[pl.ds(j * LANES, LANES)] = (
                        1.0 / jnp.maximum(tot, 1.0))
                    return carry
                lax.fori_loop(0, DEG_BUF // LANES, inv, 0)
                pltpu.sync_copy(
                    dbuf,
                    recs.at[pl.ds(sd * N_PAD + nb0 + h * DEG_BUF, DEG_BUF)])

    return body


_prop1_kernel = pl.kernel(
    _make_prop_body(True),
    out_type=(jax.ShapeDtypeStruct((2, N_PAD, EMB), _f32),
              jax.ShapeDtypeStruct((2 * N_PAD,), _f32)),
    mesh=_mesh,
    compiler_params=pltpu.CompilerParams(use_tc_tiling_on_sc=False),
    scratch_types=[
        pltpu.VMEM_SHARED((N_PAD, EMB), jnp.float32),
        pltpu.VMEM_SHARED((N_PAD,), jnp.float32),
        pltpu.VMEM((3, SG, CHUNK), jnp.int32),
        pltpu.VMEM((3, SG, CHUNK), jnp.int32),
        pltpu.VMEM((NSLOT, CHUNK, EMB), jnp.float32),
        pltpu.VMEM((CHUNK,), jnp.float32),
        pltpu.VMEM((DEG_BUF,), jnp.float32),
    ] + [pltpu.SemaphoreType.DMA] * (1 + 3 * NSLOT),
)

_prop2_kernel = pl.kernel(
    _make_prop_body(False),
    out_type=jax.ShapeDtypeStruct((2, N_PAD, EMB), _f32),
    mesh=_mesh,
    compiler_params=pltpu.CompilerParams(use_tc_tiling_on_sc=False),
    scratch_types=[
        pltpu.VMEM_SHARED((N_PAD, EMB), jnp.float32),
        pltpu.VMEM((3, SG, CHUNK), jnp.int32),
        pltpu.VMEM((3, SG, CHUNK), jnp.int32),
        pltpu.VMEM((NSLOT, CHUNK, EMB), jnp.float32),
    ] + [pltpu.SemaphoreType.DMA] * (1 + 2 * NSLOT),
)

TBLK = 2048
_TGRID = (2 * N_PAD) // TBLK


def _scale_body(x_ref, r_ref, o_ref, r32_ref):
    r32 = jnp.broadcast_to(r_ref[...], (TBLK, EMB))
    o_ref[...] = x_ref[...] * r32
    r32_ref[...] = r32


_scale_tc = pl.pallas_call(
    _scale_body,
    out_shape=(jax.ShapeDtypeStruct((2 * N_PAD, EMB), _f32),
               jax.ShapeDtypeStruct((2 * N_PAD, EMB), _f32)),
    grid=(_TGRID,),
    in_specs=[pl.BlockSpec((TBLK, EMB), lambda i: (i, 0)),
              pl.BlockSpec((TBLK, 1), lambda i: (i, 0))],
    out_specs=(pl.BlockSpec((TBLK, EMB), lambda i: (i, 0)),
               pl.BlockSpec((TBLK, EMB), lambda i: (i, 0))),
)

FBLK = 400
_FGRID = N_USER // FBLK
_FOFF = N_PAD // FBLK


def _final_body(xu_ref, xi_ref, ru_ref, ri_ref, h0u_ref, h0i_ref,
                h1u_ref, h1i_ref, ou_ref, oi_ref):
    third = _f32(1.0 / 3.0)
    ou_ref[...] = (h0u_ref[...] + h1u_ref[...]
                   + xu_ref[...] * ru_ref[...]) * third
    oi_ref[...] = (h0i_ref[...] + h1i_ref[...]
                   + xi_ref[...] * ri_ref[...]) * third


def _fspec(off, ncol):
    return pl.BlockSpec((FBLK, ncol), lambda i, off=off: (i + off, 0))


_final_tc = pl.pallas_call(
    _final_body,
    out_shape=(jax.ShapeDtypeStruct((N_USER, EMB), _f32),
               jax.ShapeDtypeStruct((N_ITEM, EMB), _f32)),
    grid=(_FGRID,),
    in_specs=[_fspec(0, EMB), _fspec(_FOFF, EMB),
              _fspec(0, EMB), _fspec(_FOFF, EMB),
              _fspec(0, EMB), _fspec(_FOFF, EMB),
              _fspec(0, EMB), _fspec(_FOFF, EMB)],
    out_specs=(_fspec(0, EMB), _fspec(0, EMB)),
)


def kernel(edge_index, user_emb, item_emb):
    pad = jnp.full((2, E_PAD - N_EDGES), PAD_NODE, jnp.int32)
    edges2 = jnp.concatenate([edge_index, pad], axis=1)
    edges2 = edges2.reshape(2, N_CHUNKS, CHUNK)
    zrow = jnp.zeros((1, N_PAD - N_USER, EMB), _f32)
    tabs0 = jnp.concatenate(
        [jnp.stack([user_emb, item_emb]),
         jnp.concatenate([zrow, zrow])], axis=1)

    sums1, recs = _prop1_kernel(edges2, tabs0)
    rec2d = recs.reshape(2 * N_PAD, 1)
    tabs1, rec32 = _scale_tc(sums1.reshape(2 * N_PAD, EMB), rec2d)
    sums2 = _prop2_kernel(edges2, tabs1.reshape(2, N_PAD, EMB))
    sums2f = sums2.reshape(2 * N_PAD, EMB)
    tabs0f = tabs0.reshape(2 * N_PAD, EMB)
    out_u, out_i = _final_tc(sums2f, sums2f, rec32, rec32,
                             tabs0f, tabs0f, tabs1, tabs1)
    return out_u, out_i


# speedup vs baseline: 1.0580x; 1.0008x over previous
import jax
import jax.numpy as jnp
from jax import lax
from jax.experimental import pallas as pl
from jax.experimental.pallas import tpu as pltpu
from jax.experimental.pallas import tpu_sc as plsc

N_USER = 50000
N_ITEM = 50000
EMB = 32
N_EDGES = 1600000

NT = 16
LANES = 16
N_PAD = 51200
NODES_PER_TILE = N_PAD // NT
CHUNK = 128
NSLOT = 4
LOOKAHEAD = 2
SG = 8
N_CHUNKS = 12544
E_PAD = N_CHUNKS * CHUNK
CHUNKS_PER_TILE = N_CHUNKS // NT
SUPERS_PER_TILE = CHUNKS_PER_TILE // SG
ITERS_PER_TILE = CHUNKS_PER_TILE // NSLOT
PAD_NODE = N_PAD - 1
DEG_BUF = NODES_PER_TILE // 2

_mesh = plsc.VectorSubcoreMesh(core_axis_name="c", subcore_axis_name="s")

_f32 = jnp.float32
_zeros16 = lambda: jnp.zeros((LANES,), _f32)


def _make_prop_body(with_deg):
    def body(*refs):
        if with_deg:
            (edges2, tabs, outs, recs,
             acc, hist, gidx3, sidx3, rows, ones_buf, dbuf,
             sem_i, *sems) = refs
            sems_g = sems[:NSLOT]
            sems_s = sems[NSLOT:2 * NSLOT]
            sems_o = sems[2 * NSLOT:]
        else:
            (edges2, tabs, outs,
             acc, gidx3, sidx3, rows,
             sem_i, *sems) = refs
            sems_g = sems[:NSLOT]
            sems_s = sems[NSLOT:]

        c = lax.axis_index("c")
        s = lax.axis_index("s")
        gd = c
        sd = 1 - c

        def zb(n, carry):
            rows[0, n, pl.ds(0, LANES)] = _zeros16()
            rows[0, n, pl.ds(LANES, LANES)] = _zeros16()
            return carry
        lax.fori_loop(0, CHUNK, zb, 0)

        nzc = NODES_PER_TILE // CHUNK
        def zacc(q, carry):
            pltpu.make_async_copy(
                rows.at[0],
                acc.at[pl.ds(s * NODES_PER_TILE + q * CHUNK, CHUNK)],
                sems_s[0]).start()
            return carry
        lax.fori_loop(0, nzc, zacc, 0)

        def zacc_wait(q, carry):
            pltpu.make_async_copy(
                rows.at[0],
                acc.at[pl.ds(s * NODES_PER_TILE + q * CHUNK, CHUNK)],
                sems_s[0]).wait()
            return carry
        lax.fori_loop(0, nzc, zacc_wait, 0)

        if with_deg:
            def zdeg(j, carry):
                dbuf[pl.ds(j * LANES, LANES)] = _zeros16()
                return carry
            lax.fori_loop(0, DEG_BUF // LANES, zdeg, 0)
            for h in range(NODES_PER_TILE // DEG_BUF):
                pltpu.sync_copy(
                    dbuf,
                    hist.at[pl.ds(s * NODES_PER_TILE + h * DEG_BUF,
                                  DEG_BUF)])

            def fill_ones(j, carry):
                ones_buf[pl.ds(j * LANES, LANES)] = jnp.ones((LANES,), _f32)
                return carry
            lax.fori_loop(0, CHUNK // LANES, fill_ones, 0)
        plsc.subcore_barrier()

        base_chunk = s * CHUNKS_PER_TILE

        def stage_desc(sg, arr, buf3):
            return pltpu.make_async_copy(
                edges2.at[arr, pl.ds(base_chunk + sg * SG, SG)],
                buf3.at[lax.rem(sg, 3)], sem_i)

        def stage(sg):
            stage_desc(sg, gd, gidx3).start()
            stage_desc(sg, sd, sidx3).start()

        def wait_stage(sg):
            stage_desc(sg, gd, gidx3).wait()
            stage_desc(sg, sd, sidx3).wait()

        def gather_desc(sgs, jof, slot):
            return pltpu.make_async_copy(
                tabs.at[gd].at[gidx3.at[sgs, jof]], rows.at[slot],
                sems_g[slot])

        def scatter_desc(sgs, jof, slot):
            return pltpu.make_async_copy(
                rows.at[slot], acc.at[sidx3.at[sgs, jof]], sems_s[slot])

        if with_deg:
            def ones_desc(sgs, jof, slot):
                return pltpu.make_async_copy(
                    ones_buf, hist.at[sidx3.at[sgs, jof]], sems_o[slot])

        def chunk_step(j, sgsm, sgs0, sgsp, do_drain, do_gather):
            slot = j % NSLOT
            gslot = (j + LOOKAHEAD) % NSLOT
            if do_drain:
                d_sgs = sgsm if j < LOOKAHEAD else sgs0
                d_jof = (j - LOOKAHEAD) % SG
                scatter_desc(d_sgs, d_jof, gslot).wait()
                if with_deg:
                    ones_desc(d_sgs, d_jof, gslot).wait()
            if do_gather:
                g_sgs = sgs0 if j < SG - LOOKAHEAD else sgsp
                g_jof = (j + LOOKAHEAD) % SG
                gather_desc(g_sgs, g_jof, gslot).start()
            gather_desc(sgs0, j, slot).wait()
            scatter_desc(sgs0, j, slot).start(add=True)
            if with_deg:
                ones_desc(sgs0, j, slot).start(add=True)

        JX = SG - LOOKAHEAD
        NS = SUPERS_PER_TILE

        pltpu.sync_copy(edges2.at[gd, pl.ds(base_chunk, SG)], gidx3.at[0])
        pltpu.sync_copy(edges2.at[sd, pl.ds(base_chunk, SG)], sidx3.at[0])
        stage(1)
        for t0 in range(LOOKAHEAD):
            gather_desc(0, t0, t0).start()

        for j in range(SG):
            if j == JX:
                wait_stage(1)
                stage(2)
            chunk_step(j, 2, 0, 1, j >= LOOKAHEAD, True)

        def outer(sg, carry):
            sgs0 = lax.rem(sg, 3)
            sgsp = lax.rem(sg + 1, 3)
            sgsm = lax.rem(sg + 2, 3)
            for j in range(SG):
                if j == JX:
                    wait_stage(sg + 1)
                    stage(sg + 2)
                chunk_step(j, sgsm, sgs0, sgsp, True, True)
            return carry
        lax.fori_loop(1, NS - 2, outer, 0)

        for j in range(SG):
            if j == JX:
                wait_stage(NS - 1)
            chunk_step(j, (NS) % 3, (NS - 2) % 3, (NS - 1) % 3, True, True)

        for j in range(SG):
            chunk_step(j, (NS - 2) % 3, (NS - 1) % 3, 0, True, j < JX)

        for j in range(SG - LOOKAHEAD, SG):
            t_tail = (NS - 1) * SG + j
            scatter_desc((NS - 1) % 3, j, t_tail % NSLOT).wait()
            if with_deg:
                ones_desc((NS - 1) % 3, j, t_tail % NSLOT).wait()
        plsc.subcore_barrier()

        nb0 = s * NODES_PER_TILE
        pltpu.sync_copy(acc.at[pl.ds(nb0, NODES_PER_TILE)],
                        outs.at[sd, pl.ds(nb0, NODES_PER_TILE)])

        if with_deg:
            for h in range(NODES_PER_TILE // DEG_BUF):
                pltpu.sync_copy(
                    hist.at[pl.ds(nb0 + h * DEG_BUF, DEG_BUF)], dbuf)

                def inv(j, carry):
                    tot = dbuf[pl.ds(j * LANES, LANES)]
                    dbuf[pl.ds(j * LANES, LANES)] = (
                        1.0 / jnp.maximum(tot, 1.0))
                    return carry
                lax.fori_loop(0, DEG_BUF // LANES, inv, 0)
                pltpu.sync_copy(
                    dbuf,
                    recs.at[pl.ds(sd * N_PAD + nb0 + h * DEG_BUF, DEG_BUF)])

    return body


_prop1_kernel = pl.kernel(
    _make_prop_body(True),
    out_type=(jax.ShapeDtypeStruct((2, N_PAD, EMB), _f32),
              jax.ShapeDtypeStruct((2 * N_PAD,), _f32)),
    mesh=_mesh,
    compiler_params=pltpu.CompilerParams(use_tc_tiling_on_sc=False),
    scratch_types=[
        pltpu.VMEM_SHARED((N_PAD, EMB), jnp.float32),
        pltpu.VMEM_SHARED((N_PAD,), jnp.float32),
        pltpu.VMEM((3, SG, CHUNK), jnp.int32),
        pltpu.VMEM((3, SG, CHUNK), jnp.int32),
        pltpu.VMEM((NSLOT, CHUNK, EMB), jnp.float32),
        pltpu.VMEM((CHUNK,), jnp.float32),
        pltpu.VMEM((DEG_BUF,), jnp.float32),
    ] + [pltpu.SemaphoreType.DMA] * (1 + 3 * NSLOT),
)

_prop2_kernel = pl.kernel(
    _make_prop_body(False),
    out_type=jax.ShapeDtypeStruct((2, N_PAD, EMB), _f32),
    mesh=_mesh,
    compiler_params=pltpu.CompilerParams(use_tc_tiling_on_sc=False),
    scratch_types=[
        pltpu.VMEM_SHARED((N_PAD, EMB), jnp.float32),
        pltpu.VMEM((3, SG, CHUNK), jnp.int32),
        pltpu.VMEM((3, SG, CHUNK), jnp.int32),
        pltpu.VMEM((NSLOT, CHUNK, EMB), jnp.float32),
    ] + [pltpu.SemaphoreType.DMA] * (1 + 2 * NSLOT),
)

TBLK = 2048
_TGRID = (2 * N_PAD) // TBLK


def _scale_body(x_ref, r_ref, o_ref, r32_ref):
    r32 = jnp.broadcast_to(r_ref[...], (TBLK, EMB))
    o_ref[...] = x_ref[...] * r32
    r32_ref[...] = r32


_scale_tc = pl.pallas_call(
    _scale_body,
    out_shape=(jax.ShapeDtypeStruct((2 * N_PAD, EMB), _f32),
               jax.ShapeDtypeStruct((2 * N_PAD, EMB), _f32)),
    grid=(_TGRID,),
    in_specs=[pl.BlockSpec((TBLK, EMB), lambda i: (i, 0)),
              pl.BlockSpec((TBLK, 1), lambda i: (i, 0))],
    out_specs=(pl.BlockSpec((TBLK, EMB), lambda i: (i, 0)),
               pl.BlockSpec((TBLK, EMB), lambda i: (i, 0))),
)

FBLK = 400
_FGRID = N_USER // FBLK
_FOFF = N_PAD // FBLK


def _final_body(xu_ref, xi_ref, ru_ref, ri_ref, h0u_ref, h0i_ref,
                h1u_ref, h1i_ref, ou_ref, oi_ref):
    third = _f32(1.0 / 3.0)
    ou_ref[...] = (h0u_ref[...] + h1u_ref[...]
                   + xu_ref[...] * ru_ref[...]) * third
    oi_ref[...] = (h0i_ref[...] + h1i_ref[...]
                   + xi_ref[...] * ri_ref[...]) * third


def _fspec(off, ncol):
    return pl.BlockSpec((FBLK, ncol), lambda i, off=off: (i + off, 0))


_final_tc = pl.pallas_call(
    _final_body,
    out_shape=(jax.ShapeDtypeStruct((N_USER, EMB), _f32),
               jax.ShapeDtypeStruct((N_ITEM, EMB), _f32)),
    grid=(_FGRID,),
    in_specs=[_fspec(0, EMB), _fspec(_FOFF, EMB),
              _fspec(0, EMB), _fspec(_FOFF, EMB),
              _fspec(0, EMB), _fspec(_FOFF, EMB),
              _fspec(0, EMB), _fspec(_FOFF, EMB)],
    out_specs=(_fspec(0, EMB), _fspec(0, EMB)),
)


def kernel(edge_index, user_emb, item_emb):
    pad = jnp.full((2, E_PAD - N_EDGES), PAD_NODE, jnp.int32)
    edges2 = jnp.concatenate([edge_index, pad], axis=1)
    edges2 = edges2.reshape(2, N_CHUNKS, CHUNK)
    zrow = jnp.zeros((1, N_PAD - N_USER, EMB), _f32)
    tabs0 = jnp.concatenate(
        [jnp.stack([user_emb, item_emb]),
         jnp.concatenate([zrow, zrow])], axis=1)

    sums1, recs = _prop1_kernel(edges2, tabs0)
    rec2d = recs.reshape(2 * N_PAD, 1)
    tabs1, rec32 = _scale_tc(sums1.reshape(2 * N_PAD, EMB), rec2d)
    sums2 = _prop2_kernel(edges2, tabs1.reshape(2, N_PAD, EMB))
    sums2f = sums2.reshape(2 * N_PAD, EMB)
    tabs0f = tabs0.reshape(2 * N_PAD, EMB)
    out_u, out_i = _final_tc(sums2f, sums2f, rec32, rec32,
                             tabs0f, tabs0f, tabs1, tabs1)
    return out_u, out_i
